# Initial kernel scaffold; baseline (speedup 1.0000x reference)
import jax
import jax.numpy as jnp
from jax.experimental import pallas as pl


def kernel(x, edge_index, W1, b1, W2, b2, lin_W, lin_b):
    raise NotImplementedError("write your pallas kernel here")


# speedup vs baseline: 26.6753x; 26.6753x over previous
import functools

import jax
import jax.numpy as jnp
from jax import lax
from jax.experimental import pallas as pl
from jax.experimental.pallas import tpu as pltpu
from jax.experimental.pallas import tpu_sc as plsc

NC = 2
NS = 16
L = 16
EB = 128
NT = NC * NS


def _sc_mesh():
    return plsc.VectorSubcoreMesh(
        core_axis_name="c", subcore_axis_name="s", num_cores=NC, num_subcores=NS
    )


def _make_deg_kernel(npad, nb):
    sl = npad // NS

    @functools.partial(
        pl.kernel,
        out_type=jax.ShapeDtypeStruct((NC, npad), jnp.float32),
        mesh=_sc_mesh(),
        scratch_types=[
            pltpu.VMEM((nb, EB), jnp.int32),
            pltpu.VMEM((EB,), jnp.float32),
            pltpu.VMEM((sl,), jnp.float32),
            pltpu.VMEM_SHARED((npad,), jnp.float32),
        ],
    )
    def deg_k(dst_hbm, out_hbm, idx_v, ones_v, z_v, deg_sp):
        c = lax.axis_index("c")
        s = lax.axis_index("s")
        t = c * NS + s
        pltpu.sync_copy(dst_hbm.at[t], idx_v)
        zero16 = jnp.zeros((L,), jnp.float32)
        one16 = jnp.ones((L,), jnp.float32)

        def zb(i, _):
            z_v[pl.ds(i * L, L)] = zero16
            return 0

        lax.fori_loop(0, sl // L, zb, 0)
        for q in range(EB // L):
            ones_v[pl.ds(q * L, L)] = one16
        pltpu.sync_copy(z_v, deg_sp.at[pl.ds(s * sl, sl)])
        plsc.subcore_barrier()

        def body(j, _):
            pltpu.sync_copy(ones_v, deg_sp.at[idx_v.at[j]], add=True)
            return 0

        lax.fori_loop(0, nb, body, 0)
        plsc.subcore_barrier()
        pltpu.sync_copy(deg_sp.at[pl.ds(s * sl, sl)], out_hbm.at[c, pl.ds(s * sl, sl)])

    return deg_k


def _make_edge_kernel(npad, nb, d):
    sl = npad // NS
    nz = sl // EB

    @functools.partial(
        pl.kernel,
        out_type=[
            jax.ShapeDtypeStruct((NC, npad, d), jnp.float32),
            jax.ShapeDtypeStruct((NC, npad), jnp.float32),
        ],
        mesh=_sc_mesh(),
        scratch_types=[
            pltpu.VMEM((nb, EB), jnp.int32),
            pltpu.VMEM((nb, EB), jnp.int32),
            pltpu.VMEM((EB, d), jnp.float32),
            pltpu.VMEM((EB,), jnp.float32),
            pltpu.VMEM_SHARED((npad, d), jnp.float32),
            pltpu.VMEM_SHARED((npad,), jnp.float32),
            pltpu.SemaphoreType.DMA,
            pltpu.SemaphoreType.DMA,
        ],
    )
    def edge_k(src_hbm, dst_hbm, g_hbm, dinv_hbm, acc_hbm, cp_hbm,
               sidx, didx, rows, cvals, acc_sp, cp_sp, sem, sem2):
        c = lax.axis_index("c")
        s = lax.axis_index("s")
        t = c * NS + s
        pltpu.sync_copy(src_hbm.at[t], sidx)
        pltpu.sync_copy(dst_hbm.at[t], didx)
        zero16 = jnp.zeros((L,), jnp.float32)

        def zb(i, _):
            for q in range(d // L):
                rows[i, pl.ds(q * L, L)] = zero16
            return 0

        lax.fori_loop(0, EB, zb, 0)
        for q in range(EB // L):
            cvals[pl.ds(q * L, L)] = zero16
        for k in range(nz):
            pltpu.sync_copy(rows, acc_sp.at[pl.ds(s * sl + k * EB, EB)])
            pltpu.sync_copy(cvals, cp_sp.at[pl.ds(s * sl + k * EB, EB)])
        plsc.subcore_barrier()

        def body(j, _):
            cg = pltpu.async_copy(dinv_hbm.at[didx.at[j]], cvals, sem2)
            pltpu.async_copy(g_hbm.at[sidx.at[j]], rows, sem).wait()
            cg.wait()
            pltpu.sync_copy(rows, acc_sp.at[didx.at[j]], add=True)
            pltpu.sync_copy(cvals, cp_sp.at[sidx.at[j]], add=True)
            return 0

        lax.fori_loop(0, nb, body, 0)
        plsc.subcore_barrier()
        pltpu.sync_copy(acc_sp.at[pl.ds(s * sl, sl)], acc_hbm.at[c, pl.ds(s * sl, sl)])
        pltpu.sync_copy(cp_sp.at[pl.ds(s * sl, sl)], cp_hbm.at[c, pl.ds(s * sl, sl)])

    return edge_k


def _mm_call(x_p, w1, degt):
    npad, _ = x_p.shape
    h = w1.shape[1]

    def body(x_ref, w_ref, degt_ref, g_ref, dinv_ref):
        deg = degt_ref[:, 0:1] + degt_ref[:, 1:2] + 1.0
        dv = lax.rsqrt(deg)
        dinv_ref[...] = dv
        hh = jnp.dot(x_ref[...], w_ref[...], preferred_element_type=jnp.float32)
        g_ref[...] = hh * dv

    return pl.pallas_call(
        body,
        out_shape=[
            jax.ShapeDtypeStruct((npad, h), jnp.float32),
            jax.ShapeDtypeStruct((npad, 1), jnp.float32),
        ],
    )(x_p, w1, degt)


def _final_call(accp, cpt, dinv_col, g, b1, w2, b2, lin_w, lin_b, n):
    npad, d = g.shape
    ncls = lin_w.shape[1]
    inv_n = 1.0 / float(n)

    def body(accp_ref, cpt_ref, dinv_ref, g_ref, b1_ref, w2_ref, b2_ref,
             lw_ref, lb_ref, out_ref):
        dv = dinv_ref[...]
        accsum = accp_ref[0] + accp_ref[1] + g_ref[...]
        h1 = jnp.maximum(dv * accsum + b1_ref[...], 0.0)
        cvec = dv * (cpt_ref[:, 0:1] + cpt_ref[:, 1:2]) + dv * dv
        rid = lax.broadcasted_iota(jnp.int32, (npad, 1), 0)
        cvec = jnp.where(rid < n, cvec, 0.0)
        v = jnp.sum(h1 * cvec, axis=0, keepdims=True) * inv_n
        pooled = jnp.dot(v, w2_ref[...], preferred_element_type=jnp.float32) + b2_ref[...]
        out_ref[...] = (
            jnp.dot(pooled, lw_ref[...], preferred_element_type=jnp.float32) + lb_ref[...]
        )

    return pl.pallas_call(
        body,
        out_shape=jax.ShapeDtypeStruct((1, ncls), jnp.float32),
    )(accp, cpt, dinv_col, g, b1.reshape(1, d), w2, b2.reshape(1, d),
      lin_w, lin_b.reshape(1, ncls))


def kernel(x, edge_index, W1, b1, W2, b2, lin_W, lin_b):
    n, dfeat = x.shape
    e = edge_index.shape[1]
    d = W1.shape[1]

    npad = ((n + 1 + NS * EB - 1) // (NS * EB)) * (NS * EB)
    nb = (e + NT * EB - 1) // (NT * EB)
    epad = NT * nb * EB

    src = edge_index[0].astype(jnp.int32)
    dst = edge_index[1].astype(jnp.int32)
    padv = jnp.full((epad - e,), n, dtype=jnp.int32)
    srcp = jnp.concatenate([src, padv]).reshape(NT, nb, EB)
    dstp = jnp.concatenate([dst, padv]).reshape(NT, nb, EB)
    x_p = jnp.pad(x, ((0, npad - n), (0, 0)))

    degp = _make_deg_kernel(npad, nb)(dstp)
    g, dinv_col = _mm_call(x_p, W1, degp.T)
    dinv_flat = dinv_col.reshape(npad)
    accp, cpp = _make_edge_kernel(npad, nb, d)(srcp, dstp, g, dinv_flat)
    return _final_call(accp, cpp.T, dinv_col, g, b1, W2, b2, lin_W, lin_b, n)
